# baseline (speedup 1.0000x reference)
import functools

import jax
import jax.numpy as jnp
from jax import lax
from jax.experimental import pallas as pl
from jax.experimental.pallas import tpu as pltpu
from jax.experimental.pallas import tpu_sc as plsc

_H = 1024
_E = 8
_F = 4096

_T = 4096
_M = 512
_P = 2 * _T + _E * _M
_NB = _P // _M
_FF = 512
_NF = _F // _FF

_G_BLK = 1024

_NC = 2
_NS = 16
_NW = _NC * _NS
_DISP_CH = 64
_CMB_CH = 64


def _gating_kernel(x_ref, gw_ref, wts_ref):
    logits = jnp.dot(x_ref[...], gw_ref[...].T,
                     preferred_element_type=jnp.float32)
    iota = jax.lax.broadcasted_iota(jnp.int32, logits.shape, 1)
    a1 = jnp.argmax(logits, axis=1)
    l1 = jnp.max(logits, axis=1, keepdims=True)
    masked = jnp.where(iota == a1[:, None], -jnp.inf, logits)
    a2 = jnp.argmax(masked, axis=1)
    l2 = jnp.max(masked, axis=1, keepdims=True)
    p1 = jax.nn.sigmoid(l1 - l2)
    wts_ref[...] = (jnp.where(iota == a1[:, None], p1, 0.0)
                    + jnp.where(iota == a2[:, None], 1.0 - p1, 0.0))


def _gmm_kernel(be_ref, act_ref, x_ref, w1_ref, b1_ref, w2_ref, b2_ref,
                wt_ref, out_ref, acc_ref):
    b = pl.program_id(0)
    ff = pl.program_id(1)

    @pl.when(act_ref[b] == 1)
    def _():
        @pl.when(ff == 0)
        def _zero():
            acc_ref[...] = jnp.zeros_like(acc_ref)

        h = jnp.dot(x_ref[...], w1_ref[0].T,
                    preferred_element_type=jnp.float32)
        h = h + b1_ref[0]
        h = 0.5 * h * (1.0 + jax.lax.erf(h * 0.7071067811865476))
        acc_ref[...] += jnp.dot(h, w2_ref[0].T,
                                preferred_element_type=jnp.float32)

        @pl.when(ff == _NF - 1)
        def _emit():
            out_ref[...] = (acc_ref[...] + b2_ref[0]) * wt_ref[:, 0:1]


def _sc_dispatch(x_hbm, tok_hbm, xs_hbm, idx_v, rows_v, sem):
    wid = lax.axis_index("s") * _NC + lax.axis_index("c")
    rpw = _P // _NW
    base = wid * rpw
    for c in range(rpw // _DISP_CH):
        off = base + c * _DISP_CH
        pltpu.sync_copy(tok_hbm.at[pl.ds(off, _DISP_CH)], idx_v)
        pltpu.async_copy(x_hbm.at[idx_v], rows_v, sem).wait()
        pltpu.sync_copy(rows_v, xs_hbm.at[pl.ds(off, _DISP_CH)])


def _sc_combine(ys_hbm, i1_hbm, i2_hbm, g1_hbm, g2_hbm, idx_v, rows_v, sem):
    wid = lax.axis_index("s") * _NC + lax.axis_index("c")
    tpw = _T // _NW
    base = wid * tpw
    for c in range(tpw // _CMB_CH):
        off = base + c * _CMB_CH
        pltpu.sync_copy(i1_hbm.at[pl.ds(off, _CMB_CH)], idx_v)
        pltpu.async_copy(ys_hbm.at[idx_v], rows_v, sem).wait()
        pltpu.sync_copy(rows_v, g1_hbm.at[pl.ds(off, _CMB_CH)])
        pltpu.sync_copy(i2_hbm.at[pl.ds(off, _CMB_CH)], idx_v)
        pltpu.async_copy(ys_hbm.at[idx_v], rows_v, sem).wait()
        pltpu.sync_copy(rows_v, g2_hbm.at[pl.ds(off, _CMB_CH)])


def _add_kernel(a_ref, b_ref, o_ref):
    o_ref[...] = a_ref[...] + b_ref[...]


def kernel(hidden_states, gate_w, w1, b1, w2, b2):
    B, S, H = hidden_states.shape
    flat = hidden_states.reshape(-1, H)
    T = flat.shape[0]
    i32 = jnp.int32

    wts = pl.pallas_call(
        _gating_kernel,
        grid=(T // _G_BLK,),
        in_specs=[
            pl.BlockSpec((_G_BLK, H), lambda m: (m, 0)),
            pl.BlockSpec((_E, H), lambda m: (0, 0)),
        ],
        out_specs=pl.BlockSpec((_G_BLK, _E), lambda m: (m, 0)),
        out_shape=jax.ShapeDtypeStruct((T, _E), jnp.float32),
    )(flat, gate_w)

    iota_e = jnp.arange(_E, dtype=i32)
    e1 = jnp.argmax(wts, axis=1).astype(i32)
    wts_m = jnp.where(iota_e[None, :] == e1[:, None], -1.0, wts)
    e2 = jnp.argmax(wts_m, axis=1).astype(i32)
    w1v = jnp.take_along_axis(wts, e1[:, None], axis=1)[:, 0]
    w2v = jnp.take_along_axis(wts, e2[:, None], axis=1)[:, 0]

    sel1 = (e1[:, None] == iota_e[None, :]).astype(i32)
    sel2 = (e2[:, None] == iota_e[None, :]).astype(i32)
    c1 = jnp.sum(sel1, axis=0)
    c2 = jnp.sum(sel2, axis=0)
    counts = c1 + c2
    padded = ((counts + _M - 1) // _M) * _M
    offs = jnp.concatenate([jnp.zeros(1, i32), jnp.cumsum(padded)[:-1]])
    total = jnp.sum(padded)

    r1 = jnp.cumsum(sel1, axis=0) - sel1
    r2 = jnp.cumsum(sel2, axis=0) - sel2
    pos1 = offs[e1] + jnp.take_along_axis(r1, e1[:, None], axis=1)[:, 0]
    pos2 = (offs[e2] + c1[e2]
            + jnp.take_along_axis(r2, e2[:, None], axis=1)[:, 0])

    tok_ids = jnp.arange(T, dtype=i32)
    tok_p = (jnp.zeros(_P, i32)
             .at[pos1].set(tok_ids, unique_indices=True)
             .at[pos2].set(tok_ids, unique_indices=True))
    wt_p = (jnp.zeros(_P, jnp.float32)
            .at[pos1].set(w1v, unique_indices=True)
            .at[pos2].set(w2v, unique_indices=True))
    wt_wide = wt_p[:, None] * jnp.ones((1, 128), jnp.float32)

    bstart = jnp.arange(_NB, dtype=i32) * _M
    be = (jnp.sum((bstart[:, None] >= offs[None, :]).astype(i32), axis=1)
          - 1).astype(i32)
    act = (bstart < total).astype(i32)

    mesh = plsc.VectorSubcoreMesh(core_axis_name="c", subcore_axis_name="s")
    xs = pl.kernel(
        _sc_dispatch,
        out_type=jax.ShapeDtypeStruct((_P, H), jnp.float32),
        mesh=mesh,
        scratch_types=[
            pltpu.VMEM((_DISP_CH,), i32),
            pltpu.VMEM((_DISP_CH, H), jnp.float32),
            pltpu.SemaphoreType.DMA,
        ],
    )(flat, tok_p)

    grid_spec = pltpu.PrefetchScalarGridSpec(
        num_scalar_prefetch=2,
        grid=(_NB, _NF),
        in_specs=[
            pl.BlockSpec((_M, H), lambda b, ff, be, act: (b, 0)),
            pl.BlockSpec((1, _FF, H), lambda b, ff, be, act: (be[b], ff, 0)),
            pl.BlockSpec((1, 1, _FF), lambda b, ff, be, act: (be[b], 0, ff)),
            pl.BlockSpec((1, H, _FF), lambda b, ff, be, act: (be[b], 0, ff)),
            pl.BlockSpec((1, 1, H), lambda b, ff, be, act: (be[b], 0, 0)),
            pl.BlockSpec((_M, 128), lambda b, ff, be, act: (b, 0)),
        ],
        out_specs=pl.BlockSpec((_M, H), lambda b, ff, be, act: (b, 0)),
        scratch_shapes=[pltpu.VMEM((_M, H), jnp.float32)],
    )
    ys = pl.pallas_call(
        _gmm_kernel,
        grid_spec=grid_spec,
        out_shape=jax.ShapeDtypeStruct((_P, H), jnp.float32),
        compiler_params=pltpu.CompilerParams(
            dimension_semantics=("arbitrary", "arbitrary"),
        ),
    )(be, act, xs, w1, b1.reshape(_E, 1, _F), w2, b2.reshape(_E, 1, H),
      wt_wide)

    g1, g2 = pl.kernel(
        _sc_combine,
        out_type=(jax.ShapeDtypeStruct((T, H), jnp.float32),
                  jax.ShapeDtypeStruct((T, H), jnp.float32)),
        mesh=plsc.VectorSubcoreMesh(core_axis_name="c",
                                    subcore_axis_name="s"),
        scratch_types=[
            pltpu.VMEM((_CMB_CH,), i32),
            pltpu.VMEM((_CMB_CH, H), jnp.float32),
            pltpu.SemaphoreType.DMA,
        ],
    )(ys, pos1, pos2)

    out = pl.pallas_call(
        _add_kernel,
        grid=(T // _G_BLK,),
        in_specs=[
            pl.BlockSpec((_G_BLK, H), lambda m: (m, 0)),
            pl.BlockSpec((_G_BLK, H), lambda m: (m, 0)),
        ],
        out_specs=pl.BlockSpec((_G_BLK, H), lambda m: (m, 0)),
        out_shape=jax.ShapeDtypeStruct((T, H), jnp.float32),
    )(g1, g2)
    return out.reshape(B, S, H)


# speedup vs baseline: 1.0332x; 1.0332x over previous
import functools

import jax
import jax.numpy as jnp
from jax import lax
from jax.experimental import pallas as pl
from jax.experimental.pallas import tpu as pltpu
from jax.experimental.pallas import tpu_sc as plsc

_H = 1024
_E = 8
_F = 4096

_T = 4096
_M = 256
_P = 2 * _T + _E * _M
_NB = _P // _M
_FF = 512
_NF = _F // _FF

_G_BLK = 1024

_NC = 2
_NS = 16
_NW = _NC * _NS
_DISP_CH = 32
_DISP_DEPTH = 3
_CMB_CH = 64


def _gating_kernel(x_ref, gw_ref, wts_ref):
    logits = jnp.dot(x_ref[...], gw_ref[...].T,
                     preferred_element_type=jnp.float32)
    iota = jax.lax.broadcasted_iota(jnp.int32, logits.shape, 1)
    a1 = jnp.argmax(logits, axis=1)
    l1 = jnp.max(logits, axis=1, keepdims=True)
    masked = jnp.where(iota == a1[:, None], -jnp.inf, logits)
    a2 = jnp.argmax(masked, axis=1)
    l2 = jnp.max(masked, axis=1, keepdims=True)
    p1 = jax.nn.sigmoid(l1 - l2)
    wts_ref[...] = (jnp.where(iota == a1[:, None], p1, 0.0)
                    + jnp.where(iota == a2[:, None], 1.0 - p1, 0.0))


def _gmm_kernel(be_ref, act_ref, x_ref, w1_ref, b1_ref, w2_ref, b2_ref,
                wt_ref, out_ref, acc_ref):
    b = pl.program_id(0)
    ff = pl.program_id(1)

    @pl.when(act_ref[b] == 1)
    def _():
        @pl.when(ff == 0)
        def _zero():
            acc_ref[...] = jnp.zeros_like(acc_ref)

        h = jnp.dot(x_ref[...], w1_ref[0].T,
                    preferred_element_type=jnp.float32)
        h = h + b1_ref[0]
        h = 0.5 * h * (1.0 + jax.lax.erf(h * 0.7071067811865476))
        acc_ref[...] += jnp.dot(h, w2_ref[0].T,
                                preferred_element_type=jnp.float32)

        @pl.when(ff == _NF - 1)
        def _emit():
            out_ref[...] = (acc_ref[...] + b2_ref[0]) * wt_ref[:, 0:1]


def _sc_dispatch(x_hbm, tok_hbm, xs_hbm, idx_v, rows0, rows1, rows2,
                 gs0, gs1, gs2, ws0, ws1, ws2):
    wid = lax.axis_index("s") * _NC + lax.axis_index("c")
    rpw = _P // _NW
    base = wid * rpw
    nch = rpw // _DISP_CH
    rows = (rows0, rows1, rows2)
    gsem = (gs0, gs1, gs2)
    wsem = (ws0, ws1, ws2)
    pltpu.sync_copy(tok_hbm.at[pl.ds(base, rpw)], idx_v)

    def gather(c):
        return pltpu.async_copy(
            x_hbm.at[idx_v.at[pl.ds(c * _DISP_CH, _DISP_CH)]],
            rows[c % _DISP_DEPTH], gsem[c % _DISP_DEPTH])

    g = {}
    w = {}
    for c in range(min(_DISP_DEPTH, nch)):
        g[c] = gather(c)
    for c in range(nch):
        g[c].wait()
        w[c] = pltpu.async_copy(
            rows[c % _DISP_DEPTH],
            xs_hbm.at[pl.ds(base + c * _DISP_CH, _DISP_CH)],
            wsem[c % _DISP_DEPTH])
        if c + _DISP_DEPTH < nch:
            w[c].wait()
            g[c + _DISP_DEPTH] = gather(c + _DISP_DEPTH)
    for c in range(max(0, nch - _DISP_DEPTH), nch):
        w[c].wait()


def _sc_combine(ys_hbm, i1_hbm, i2_hbm, g1_hbm, g2_hbm, idx_v, rows_v, sem):
    wid = lax.axis_index("s") * _NC + lax.axis_index("c")
    tpw = _T // _NW
    base = wid * tpw
    for c in range(tpw // _CMB_CH):
        off = base + c * _CMB_CH
        pltpu.sync_copy(i1_hbm.at[pl.ds(off, _CMB_CH)], idx_v)
        pltpu.async_copy(ys_hbm.at[idx_v], rows_v, sem).wait()
        pltpu.sync_copy(rows_v, g1_hbm.at[pl.ds(off, _CMB_CH)])
        pltpu.sync_copy(i2_hbm.at[pl.ds(off, _CMB_CH)], idx_v)
        pltpu.async_copy(ys_hbm.at[idx_v], rows_v, sem).wait()
        pltpu.sync_copy(rows_v, g2_hbm.at[pl.ds(off, _CMB_CH)])


def _add_kernel(a_ref, b_ref, o_ref):
    o_ref[...] = a_ref[...] + b_ref[...]


def kernel(hidden_states, gate_w, w1, b1, w2, b2):
    B, S, H = hidden_states.shape
    flat = hidden_states.reshape(-1, H)
    T = flat.shape[0]
    i32 = jnp.int32

    wts = pl.pallas_call(
        _gating_kernel,
        grid=(T // _G_BLK,),
        in_specs=[
            pl.BlockSpec((_G_BLK, H), lambda m: (m, 0)),
            pl.BlockSpec((_E, H), lambda m: (0, 0)),
        ],
        out_specs=pl.BlockSpec((_G_BLK, _E), lambda m: (m, 0)),
        out_shape=jax.ShapeDtypeStruct((T, _E), jnp.float32),
    )(flat, gate_w)

    iota_e = jnp.arange(_E, dtype=i32)
    e1 = jnp.argmax(wts, axis=1).astype(i32)
    wts_m = jnp.where(iota_e[None, :] == e1[:, None], -1.0, wts)
    e2 = jnp.argmax(wts_m, axis=1).astype(i32)
    w1v = jnp.take_along_axis(wts, e1[:, None], axis=1)[:, 0]
    w2v = jnp.take_along_axis(wts, e2[:, None], axis=1)[:, 0]

    sel1 = (e1[:, None] == iota_e[None, :]).astype(i32)
    sel2 = (e2[:, None] == iota_e[None, :]).astype(i32)
    c1 = jnp.sum(sel1, axis=0)
    c2 = jnp.sum(sel2, axis=0)
    counts = c1 + c2
    padded = ((counts + _M - 1) // _M) * _M
    offs = jnp.concatenate([jnp.zeros(1, i32), jnp.cumsum(padded)[:-1]])
    total = jnp.sum(padded)

    r1 = jnp.cumsum(sel1, axis=0) - sel1
    r2 = jnp.cumsum(sel2, axis=0) - sel2
    pos1 = offs[e1] + jnp.take_along_axis(r1, e1[:, None], axis=1)[:, 0]
    pos2 = (offs[e2] + c1[e2]
            + jnp.take_along_axis(r2, e2[:, None], axis=1)[:, 0])

    tok_ids = jnp.arange(T, dtype=i32)
    tok_p = ((jnp.arange(_P, dtype=i32) % T)
             .at[pos1].set(tok_ids, unique_indices=True)
             .at[pos2].set(tok_ids, unique_indices=True))
    wt_p = (jnp.zeros(_P, jnp.float32)
            .at[pos1].set(w1v, unique_indices=True)
            .at[pos2].set(w2v, unique_indices=True))
    wt_wide = wt_p[:, None] * jnp.ones((1, 128), jnp.float32)

    bstart = jnp.arange(_NB, dtype=i32) * _M
    be = (jnp.sum((bstart[:, None] >= offs[None, :]).astype(i32), axis=1)
          - 1).astype(i32)
    act = (bstart < total).astype(i32)

    mesh = plsc.VectorSubcoreMesh(core_axis_name="c", subcore_axis_name="s")
    xs = pl.kernel(
        _sc_dispatch,
        out_type=jax.ShapeDtypeStruct((_P, H), jnp.float32),
        mesh=mesh,
        scratch_types=[
            pltpu.VMEM((_P // _NW,), i32),
            pltpu.VMEM((_DISP_CH, H), jnp.float32),
            pltpu.VMEM((_DISP_CH, H), jnp.float32),
            pltpu.VMEM((_DISP_CH, H), jnp.float32),
            pltpu.SemaphoreType.DMA,
            pltpu.SemaphoreType.DMA,
            pltpu.SemaphoreType.DMA,
            pltpu.SemaphoreType.DMA,
            pltpu.SemaphoreType.DMA,
            pltpu.SemaphoreType.DMA,
        ],
    )(flat, tok_p)

    grid_spec = pltpu.PrefetchScalarGridSpec(
        num_scalar_prefetch=2,
        grid=(_NB, _NF),
        in_specs=[
            pl.BlockSpec((_M, H), lambda b, ff, be, act: (b, 0)),
            pl.BlockSpec((1, _FF, H), lambda b, ff, be, act: (be[b], ff, 0)),
            pl.BlockSpec((1, 1, _FF), lambda b, ff, be, act: (be[b], 0, ff)),
            pl.BlockSpec((1, H, _FF), lambda b, ff, be, act: (be[b], 0, ff)),
            pl.BlockSpec((1, 1, H), lambda b, ff, be, act: (be[b], 0, 0)),
            pl.BlockSpec((_M, 128), lambda b, ff, be, act: (b, 0)),
        ],
        out_specs=pl.BlockSpec((_M, H), lambda b, ff, be, act: (b, 0)),
        scratch_shapes=[pltpu.VMEM((_M, H), jnp.float32)],
    )
    ys = pl.pallas_call(
        _gmm_kernel,
        grid_spec=grid_spec,
        out_shape=jax.ShapeDtypeStruct((_P, H), jnp.float32),
        compiler_params=pltpu.CompilerParams(
            dimension_semantics=("arbitrary", "arbitrary"),
        ),
    )(be, act, xs, w1, b1.reshape(_E, 1, _F), w2, b2.reshape(_E, 1, H),
      wt_wide)

    g1, g2 = pl.kernel(
        _sc_combine,
        out_type=(jax.ShapeDtypeStruct((T, H), jnp.float32),
                  jax.ShapeDtypeStruct((T, H), jnp.float32)),
        mesh=plsc.VectorSubcoreMesh(core_axis_name="c",
                                    subcore_axis_name="s"),
        scratch_types=[
            pltpu.VMEM((_CMB_CH,), i32),
            pltpu.VMEM((_CMB_CH, H), jnp.float32),
            pltpu.SemaphoreType.DMA,
        ],
    )(ys, pos1, pos2)

    out = pl.pallas_call(
        _add_kernel,
        grid=(T // _G_BLK,),
        in_specs=[
            pl.BlockSpec((_G_BLK, H), lambda m: (m, 0)),
            pl.BlockSpec((_G_BLK, H), lambda m: (m, 0)),
        ],
        out_specs=pl.BlockSpec((_G_BLK, H), lambda m: (m, 0)),
        out_shape=jax.ShapeDtypeStruct((T, H), jnp.float32),
    )(g1, g2)
    return out.reshape(B, S, H)


# speedup vs baseline: 1.3762x; 1.3320x over previous
import functools

import jax
import jax.numpy as jnp
from jax import lax
from jax.experimental import pallas as pl
from jax.experimental.pallas import tpu as pltpu
from jax.experimental.pallas import tpu_sc as plsc

_H = 1024
_E = 8
_F = 4096

_T = 4096
_M = 512
_P = 2 * _T + _E * _M
_NB = _P // _M
_FF = 512
_NF = _F // _FF

_G_BLK = 1024

_NC = 2
_NS = 16
_NW = _NC * _NS
_DISP_CH = 32
_DISP_DEPTH = 3
_CMB_CH = 64


def _gating_kernel(x_ref, gw_ref, wts_ref):
    logits = jnp.dot(x_ref[...], gw_ref[...].T,
                     preferred_element_type=jnp.float32)
    iota = jax.lax.broadcasted_iota(jnp.int32, logits.shape, 1)
    a1 = jnp.argmax(logits, axis=1)
    l1 = jnp.max(logits, axis=1, keepdims=True)
    masked = jnp.where(iota == a1[:, None], -jnp.inf, logits)
    a2 = jnp.argmax(masked, axis=1)
    l2 = jnp.max(masked, axis=1, keepdims=True)
    p1 = jax.nn.sigmoid(l1 - l2)
    wts_ref[...] = (jnp.where(iota == a1[:, None], p1, 0.0)
                    + jnp.where(iota == a2[:, None], 1.0 - p1, 0.0))


def _gmm_kernel(be_ref, act_ref, x_ref, w1_ref, b1_ref, w2_ref, b2_ref,
                wt_ref, out_ref, acc_ref):
    b = pl.program_id(0)
    ff = pl.program_id(1)

    @pl.when(act_ref[b] == 1)
    def _():
        @pl.when(ff == 0)
        def _zero():
            acc_ref[...] = jnp.zeros_like(acc_ref)

        h = jnp.dot(x_ref[...], w1_ref[0].T,
                    preferred_element_type=jnp.float32)
        h = h + b1_ref[0]
        h = 0.5 * h * (1.0 + jax.lax.erf(h * 0.7071067811865476))
        acc_ref[...] += jnp.dot(h, w2_ref[0].T,
                                preferred_element_type=jnp.float32)

        @pl.when(ff == _NF - 1)
        def _emit():
            out_ref[...] = (acc_ref[...] + b2_ref[0]) * wt_ref[:, 0:1]


def _sc_dispatch(x_hbm, tok_hbm, xs_hbm, idx_v, rows0, rows1, rows2,
                 gs0, gs1, gs2, ws0, ws1, ws2):
    wid = lax.axis_index("s") * _NC + lax.axis_index("c")
    rpw = _P // _NW
    base = wid * rpw
    nch = rpw // _DISP_CH
    rows = (rows0, rows1, rows2)
    gsem = (gs0, gs1, gs2)
    wsem = (ws0, ws1, ws2)
    pltpu.sync_copy(tok_hbm.at[pl.ds(base, rpw)], idx_v)

    def gather(c):
        return pltpu.async_copy(
            x_hbm.at[idx_v.at[pl.ds(c * _DISP_CH, _DISP_CH)]],
            rows[c % _DISP_DEPTH], gsem[c % _DISP_DEPTH])

    g = {}
    w = {}
    for c in range(min(_DISP_DEPTH, nch)):
        g[c] = gather(c)
    for c in range(nch):
        g[c].wait()
        w[c] = pltpu.async_copy(
            rows[c % _DISP_DEPTH],
            xs_hbm.at[pl.ds(base + c * _DISP_CH, _DISP_CH)],
            wsem[c % _DISP_DEPTH])
        if c + _DISP_DEPTH < nch:
            w[c].wait()
            g[c + _DISP_DEPTH] = gather(c + _DISP_DEPTH)
    for c in range(max(0, nch - _DISP_DEPTH), nch):
        w[c].wait()


def _sc_combine(ys_hbm, i1_hbm, i2_hbm, g1_hbm, g2_hbm, idx_v, rows_v, sem):
    wid = lax.axis_index("s") * _NC + lax.axis_index("c")
    tpw = _T // _NW
    base = wid * tpw
    for c in range(tpw // _CMB_CH):
        off = base + c * _CMB_CH
        pltpu.sync_copy(i1_hbm.at[pl.ds(off, _CMB_CH)], idx_v)
        pltpu.async_copy(ys_hbm.at[idx_v], rows_v, sem).wait()
        pltpu.sync_copy(rows_v, g1_hbm.at[pl.ds(off, _CMB_CH)])
        pltpu.sync_copy(i2_hbm.at[pl.ds(off, _CMB_CH)], idx_v)
        pltpu.async_copy(ys_hbm.at[idx_v], rows_v, sem).wait()
        pltpu.sync_copy(rows_v, g2_hbm.at[pl.ds(off, _CMB_CH)])


def _add_kernel(a_ref, b_ref, o_ref):
    o_ref[...] = a_ref[...] + b_ref[...]


def kernel(hidden_states, gate_w, w1, b1, w2, b2):
    B, S, H = hidden_states.shape
    flat = hidden_states.reshape(-1, H)
    T = flat.shape[0]
    i32 = jnp.int32

    wts = pl.pallas_call(
        _gating_kernel,
        grid=(T // _G_BLK,),
        in_specs=[
            pl.BlockSpec((_G_BLK, H), lambda m: (m, 0)),
            pl.BlockSpec((_E, H), lambda m: (0, 0)),
        ],
        out_specs=pl.BlockSpec((_G_BLK, _E), lambda m: (m, 0)),
        out_shape=jax.ShapeDtypeStruct((T, _E), jnp.float32),
    )(flat, gate_w)

    iota_e = jnp.arange(_E, dtype=i32)
    e1 = jnp.argmax(wts, axis=1).astype(i32)
    wts_m = jnp.where(iota_e[None, :] == e1[:, None], -1.0, wts)
    e2 = jnp.argmax(wts_m, axis=1).astype(i32)
    w1v = jnp.take_along_axis(wts, e1[:, None], axis=1)[:, 0]
    w2v = jnp.take_along_axis(wts, e2[:, None], axis=1)[:, 0]

    sel1 = (e1[:, None] == iota_e[None, :]).astype(i32)
    sel2 = (e2[:, None] == iota_e[None, :]).astype(i32)
    c1 = jnp.sum(sel1, axis=0)
    c2 = jnp.sum(sel2, axis=0)
    counts = c1 + c2
    padded = ((counts + _M - 1) // _M) * _M
    offs = jnp.concatenate([jnp.zeros(1, i32), jnp.cumsum(padded)[:-1]])
    total = jnp.sum(padded)

    r1 = jnp.cumsum(sel1, axis=0) - sel1
    r2 = jnp.cumsum(sel2, axis=0) - sel2
    pos1 = offs[e1] + jnp.take_along_axis(r1, e1[:, None], axis=1)[:, 0]
    pos2 = (offs[e2] + c1[e2]
            + jnp.take_along_axis(r2, e2[:, None], axis=1)[:, 0])

    tok_ids = jnp.arange(T, dtype=i32)
    tok_p = ((jnp.arange(_P, dtype=i32) % T)
             .at[pos1].set(tok_ids, unique_indices=True)
             .at[pos2].set(tok_ids, unique_indices=True))
    wt_p = (jnp.zeros(_P, jnp.float32)
            .at[pos1].set(w1v, unique_indices=True)
            .at[pos2].set(w2v, unique_indices=True))
    wt_wide = wt_p[:, None] * jnp.ones((1, 128), jnp.float32)

    bstart = jnp.arange(_NB, dtype=i32) * _M
    be = (jnp.sum((bstart[:, None] >= offs[None, :]).astype(i32), axis=1)
          - 1).astype(i32)
    act = (bstart < total).astype(i32)

    mesh = plsc.VectorSubcoreMesh(core_axis_name="c", subcore_axis_name="s")
    xs = pl.kernel(
        _sc_dispatch,
        out_type=jax.ShapeDtypeStruct((_P, H), jnp.float32),
        mesh=mesh,
        scratch_types=[
            pltpu.VMEM((_P // _NW,), i32),
            pltpu.VMEM((_DISP_CH, H), jnp.float32),
            pltpu.VMEM((_DISP_CH, H), jnp.float32),
            pltpu.VMEM((_DISP_CH, H), jnp.float32),
            pltpu.SemaphoreType.DMA,
            pltpu.SemaphoreType.DMA,
            pltpu.SemaphoreType.DMA,
            pltpu.SemaphoreType.DMA,
            pltpu.SemaphoreType.DMA,
            pltpu.SemaphoreType.DMA,
        ],
    )(flat, tok_p)

    grid_spec = pltpu.PrefetchScalarGridSpec(
        num_scalar_prefetch=2,
        grid=(_NB, _NF),
        in_specs=[
            pl.BlockSpec((_M, H), lambda b, ff, be, act: (b, 0)),
            pl.BlockSpec((1, _FF, H), lambda b, ff, be, act: (be[b], ff, 0)),
            pl.BlockSpec((1, 1, _FF), lambda b, ff, be, act: (be[b], 0, ff)),
            pl.BlockSpec((1, H, _FF), lambda b, ff, be, act: (be[b], 0, ff)),
            pl.BlockSpec((1, 1, H), lambda b, ff, be, act: (be[b], 0, 0)),
            pl.BlockSpec((_M, 128), lambda b, ff, be, act: (b, 0)),
        ],
        out_specs=pl.BlockSpec((_M, H), lambda b, ff, be, act: (b, 0)),
        scratch_shapes=[pltpu.VMEM((_M, H), jnp.float32)],
    )
    ys = pl.pallas_call(
        _gmm_kernel,
        grid_spec=grid_spec,
        out_shape=jax.ShapeDtypeStruct((_P, H), jnp.float32),
        compiler_params=pltpu.CompilerParams(
            dimension_semantics=("arbitrary", "arbitrary"),
        ),
    )(be, act, xs, w1, b1.reshape(_E, 1, _F), w2, b2.reshape(_E, 1, H),
      wt_wide)

    g1, g2 = pl.kernel(
        _sc_combine,
        out_type=(jax.ShapeDtypeStruct((T, H), jnp.float32),
                  jax.ShapeDtypeStruct((T, H), jnp.float32)),
        mesh=plsc.VectorSubcoreMesh(core_axis_name="c",
                                    subcore_axis_name="s"),
        scratch_types=[
            pltpu.VMEM((_CMB_CH,), i32),
            pltpu.VMEM((_CMB_CH, H), jnp.float32),
            pltpu.SemaphoreType.DMA,
        ],
    )(ys, pos1, pos2)

    out = pl.pallas_call(
        _add_kernel,
        grid=(T // _G_BLK,),
        in_specs=[
            pl.BlockSpec((_G_BLK, H), lambda m: (m, 0)),
            pl.BlockSpec((_G_BLK, H), lambda m: (m, 0)),
        ],
        out_specs=pl.BlockSpec((_G_BLK, H), lambda m: (m, 0)),
        out_shape=jax.ShapeDtypeStruct((T, H), jnp.float32),
    )(g1, g2)
    return out.reshape(B, S, H)
